# Initial kernel scaffold; baseline (speedup 1.0000x reference)
import jax
import jax.numpy as jnp
from jax.experimental import pallas as pl


def kernel(inputs, weight):
    raise NotImplementedError("write your pallas kernel here")


# speedup vs baseline: 5.4846x; 5.4846x over previous
import jax
import jax.numpy as jnp
from jax.experimental import pallas as pl
from jax.experimental.pallas import tpu as pltpu

_K = 8192
_D = 256
_BM = 256
_B1 = 2736
_B2 = 5472
_BETA = 0.25


def _seg_min(d, cols, lo, hi):
    big = jnp.float32(jnp.inf)
    mask = (cols >= lo) & (cols < hi)
    dm = jnp.where(mask, d, big)
    m = jnp.min(dm, axis=1, keepdims=True)
    i = jnp.min(jnp.where(dm == m, cols, _K), axis=1, keepdims=True)
    return m, i


def _vq_block(x_ref, w_ref, x2_ref, w2_ref, enc_ref, idx_ref, dsel_ref, q_ref):
    x = x_ref[...]
    w = w_ref[...]
    xb = x.astype(jnp.bfloat16)
    wb = w.astype(jnp.bfloat16)
    t = jax.lax.dot_general(
        xb, wb, (((1,), (1,)), ((), ())),
        preferred_element_type=jnp.float32)
    d = (x2_ref[...] + w2_ref[...]) - 2.0 * t
    cols = jax.lax.broadcasted_iota(jnp.int32, d.shape, 1)
    m0, i0 = _seg_min(d, cols, 0, _B1)
    m1, i1 = _seg_min(d, cols, _B1, _B2)
    m2, i2 = _seg_min(d, cols, _B2, _K)
    b0 = m0.astype(jnp.bfloat16).astype(jnp.float32)
    win1 = m1 < b0
    v1 = jnp.where(win1, m1, m0)
    j1 = jnp.where(win1, i1, i0)
    b1 = v1.astype(jnp.bfloat16).astype(jnp.float32)
    win2 = m2 < b1
    dsel = jnp.where(win2, m2, v1)
    idx = jnp.where(win2, i2, j1)
    enc = (cols == idx).astype(jnp.float32)
    enc_ref[...] = enc
    idx_ref[...] = idx
    dsel_ref[...] = dsel
    q_ref[...] = jax.lax.dot_general(
        enc.astype(jnp.bfloat16), wb, (((1,), (0,)), ((), ())),
        preferred_element_type=jnp.float32)


def kernel(inputs, weight):
    n, c, h, wd = inputs.shape
    x = jnp.transpose(inputs, (0, 2, 3, 1))
    flat = x.reshape(-1, _D)
    nrows = flat.shape[0]
    x2 = jnp.sum(flat ** 2, axis=1, keepdims=True)
    w2 = jnp.sum(weight ** 2, axis=1)[None, :]
    grid = (nrows // _BM,)
    enc, idx, dsel, q = pl.pallas_call(
        _vq_block,
        grid=grid,
        in_specs=[
            pl.BlockSpec((_BM, _D), lambda i: (i, 0)),
            pl.BlockSpec((_K, _D), lambda i: (0, 0)),
            pl.BlockSpec((_BM, 1), lambda i: (i, 0)),
            pl.BlockSpec((1, _K), lambda i: (0, 0)),
        ],
        out_specs=[
            pl.BlockSpec((_BM, _K), lambda i: (i, 0)),
            pl.BlockSpec((_BM, 1), lambda i: (i, 0)),
            pl.BlockSpec((_BM, 1), lambda i: (i, 0)),
            pl.BlockSpec((_BM, _D), lambda i: (i, 0)),
        ],
        out_shape=[
            jax.ShapeDtypeStruct((nrows, _K), jnp.float32),
            jax.ShapeDtypeStruct((nrows, 1), jnp.int32),
            jax.ShapeDtypeStruct((nrows, 1), jnp.float32),
            jax.ShapeDtypeStruct((nrows, _D), jnp.float32),
        ],
    )(flat, weight, x2, w2)
    m = jnp.sum(dsel) / (nrows * _D)
    loss = m + _BETA * m
    quantized_nchw = jnp.transpose(q.reshape(n, h, wd, c), (0, 3, 1, 2))
    return (loss, quantized_nchw, enc, idx)
